# baseline (device time: 65399 ns/iter reference)
import functools

import jax
import jax.numpy as jnp
from jax import lax
from jax.experimental import pallas as pl
from jax.experimental.pallas import tpu as pltpu

N_DEV = 4
TBLK = 8
CORR_T = 64


def kernel(x, A, B, C):
    Bb, L, D = x.shape
    N = A.shape[1]
    At = A.T

    def body(x_ref, at_ref, b_ref, c_ref, out_ref,
             carry_ref, hin_ref, send_sem, recv_sem):
        my = lax.axis_index("i")
        left = (my + N_DEV - 1) % N_DEV
        right = (my + 1) % N_DEV

        barrier_sem = pltpu.get_barrier_semaphore()
        pl.semaphore_signal(barrier_sem, inc=1, device_id=(left,),
                            device_id_type=pl.DeviceIdType.MESH)
        pl.semaphore_wait(barrier_sem, 1)

        dAt = jnp.exp(at_ref[...])[None]

        def blk_step(i, h):
            t0 = i * TBLK
            xblk = x_ref[:, pl.ds(t0, TBLK), :]
            bblk = b_ref[:, pl.ds(t0, TBLK), :]
            cblk = c_ref[:, pl.ds(t0, TBLK), :]
            ys = []
            for j in range(TBLK):
                xt = xblk[:, j, :]
                bt = bblk[:, j, :]
                ct = cblk[:, j, :]
                h = h * dAt + xt[:, None, :] * bt[:, :, None]
                ys.append(jnp.sum(h * ct[:, :, None], axis=1))
            out_ref[:, pl.ds(t0, TBLK), :] = jnp.stack(ys, axis=1)
            return h

        h0 = jnp.zeros((Bb, N, D), jnp.float32)
        h_final = lax.fori_loop(0, L // TBLK, blk_step, h0)

        rdma = pltpu.make_async_remote_copy(
            src_ref=carry_ref, dst_ref=hin_ref,
            send_sem=send_sem, recv_sem=recv_sem,
            device_id=(right,), device_id_type=pl.DeviceIdType.MESH)

        @pl.when(my < N_DEV - 1)
        def _():
            carry_ref[...] = h_final
            rdma.start()

        @pl.when(my > 0)
        def _():
            rdma.wait_recv()

            def corr_step(i, hc):
                t0 = i * TBLK
                cblk = c_ref[:, pl.ds(t0, TBLK), :]
                yblk = out_ref[:, pl.ds(t0, TBLK), :]
                ys = []
                for j in range(TBLK):
                    hc = hc * dAt
                    ys.append(jnp.sum(hc * cblk[:, j, :][:, :, None], axis=1))
                out_ref[:, pl.ds(t0, TBLK), :] = yblk + jnp.stack(ys, axis=1)
                return hc

            lax.fori_loop(0, CORR_T // TBLK, corr_step, hin_ref[...])

        @pl.when(my < N_DEV - 1)
        def _():
            rdma.wait_send()

        @functools.partial(pl.run_scoped,
                           exit_sem=pltpu.SemaphoreType.REGULAR)
        def _(exit_sem):
            pl.semaphore_signal(exit_sem, inc=1, device_id=(left,),
                                device_id_type=pl.DeviceIdType.MESH)
            pl.semaphore_wait(exit_sem, 1)

    return pl.pallas_call(
        body,
        out_shape=jax.ShapeDtypeStruct((Bb, L, D), jnp.float32),
        in_specs=[
            pl.BlockSpec(memory_space=pltpu.VMEM),
            pl.BlockSpec(memory_space=pltpu.VMEM),
            pl.BlockSpec(memory_space=pltpu.VMEM),
            pl.BlockSpec(memory_space=pltpu.VMEM),
        ],
        out_specs=pl.BlockSpec(memory_space=pltpu.VMEM),
        scratch_shapes=[
            pltpu.VMEM((Bb, N, D), jnp.float32),
            pltpu.VMEM((Bb, N, D), jnp.float32),
            pltpu.SemaphoreType.DMA,
            pltpu.SemaphoreType.DMA,
        ],
        compiler_params=pltpu.CompilerParams(collective_id=0),
    )(x, At, B, C)


# device time: 54787 ns/iter; 1.1937x vs baseline; 1.1937x over previous
import functools

import jax
import jax.numpy as jnp
from jax import lax
from jax.experimental import pallas as pl
from jax.experimental.pallas import tpu as pltpu

N_DEV = 4
TBLK = 8
CORR_T = 64


def kernel(x, A, B, C):
    Bb, L, D = x.shape
    N = A.shape[1]
    At = A.T

    def body(x_ref, at_ref, b_ref, c_ref, out_ref,
             carry_ref, hin_ref, send_sem, recv_sem):
        my = lax.axis_index("i")
        left = (my + N_DEV - 1) % N_DEV
        right = (my + 1) % N_DEV

        barrier_sem = pltpu.get_barrier_semaphore()
        pl.semaphore_signal(barrier_sem, inc=1, device_id=(left,),
                            device_id_type=pl.DeviceIdType.MESH)
        pl.semaphore_wait(barrier_sem, 1)

        dAt = jnp.exp(at_ref[...])[None]

        def blk_step(i, h):
            t0 = i * TBLK
            xblk = x_ref[:, pl.ds(t0, TBLK), :]
            bblk = b_ref[:, pl.ds(t0, TBLK), :]
            cblk = c_ref[:, pl.ds(t0, TBLK), :]
            ys = []
            for j in range(TBLK):
                xt = xblk[:, j, :]
                bt = bblk[:, j, :]
                ct = cblk[:, j, :]
                h = h * dAt + xt[:, None, :] * bt[:, :, None]
                ys.append(lax.dot_general(
                    ct, h, (((1,), (1,)), ((0,), (0,))),
                    preferred_element_type=jnp.float32))
            out_ref[:, pl.ds(t0, TBLK), :] = jnp.stack(ys, axis=1)
            return h

        h0 = jnp.zeros((Bb, N, D), jnp.float32)
        h_final = lax.fori_loop(0, L // TBLK, blk_step, h0)

        rdma = pltpu.make_async_remote_copy(
            src_ref=carry_ref, dst_ref=hin_ref,
            send_sem=send_sem, recv_sem=recv_sem,
            device_id=(right,), device_id_type=pl.DeviceIdType.MESH)

        @pl.when(my < N_DEV - 1)
        def _():
            carry_ref[...] = h_final
            rdma.start()

        @pl.when(my > 0)
        def _():
            rdma.wait_recv()

            def corr_step(i, hc):
                t0 = i * TBLK
                cblk = c_ref[:, pl.ds(t0, TBLK), :]
                yblk = out_ref[:, pl.ds(t0, TBLK), :]
                ys = []
                for j in range(TBLK):
                    hc = hc * dAt
                    ys.append(lax.dot_general(
                        cblk[:, j, :], hc, (((1,), (1,)), ((0,), (0,))),
                        preferred_element_type=jnp.float32))
                out_ref[:, pl.ds(t0, TBLK), :] = yblk + jnp.stack(ys, axis=1)
                return hc

            lax.fori_loop(0, CORR_T // TBLK, corr_step, hin_ref[...])

        @pl.when(my < N_DEV - 1)
        def _():
            rdma.wait_send()

        @functools.partial(pl.run_scoped,
                           exit_sem=pltpu.SemaphoreType.REGULAR)
        def _(exit_sem):
            pl.semaphore_signal(exit_sem, inc=1, device_id=(left,),
                                device_id_type=pl.DeviceIdType.MESH)
            pl.semaphore_wait(exit_sem, 1)

    return pl.pallas_call(
        body,
        out_shape=jax.ShapeDtypeStruct((Bb, L, D), jnp.float32),
        in_specs=[
            pl.BlockSpec(memory_space=pltpu.VMEM),
            pl.BlockSpec(memory_space=pltpu.VMEM),
            pl.BlockSpec(memory_space=pltpu.VMEM),
            pl.BlockSpec(memory_space=pltpu.VMEM),
        ],
        out_specs=pl.BlockSpec(memory_space=pltpu.VMEM),
        scratch_shapes=[
            pltpu.VMEM((Bb, N, D), jnp.float32),
            pltpu.VMEM((Bb, N, D), jnp.float32),
            pltpu.SemaphoreType.DMA,
            pltpu.SemaphoreType.DMA,
        ],
        compiler_params=pltpu.CompilerParams(collective_id=0),
    )(x, At, B, C)
